# Initial kernel scaffold; baseline (speedup 1.0000x reference)
import jax
import jax.numpy as jnp
from jax.experimental import pallas as pl


def kernel(x, edge_index, edge_attr, Wl, bl, Wr, br, We, att, bias, W_ea, b_ea, gn_weight, gn_bias, gn_mean_scale):
    raise NotImplementedError("write your pallas kernel here")


# speedup vs baseline: 40.6256x; 40.6256x over previous
import jax
import jax.numpy as jnp
from jax import lax
from jax.experimental import pallas as pl
from jax.experimental.pallas import tpu as pltpu
from jax.experimental.pallas import tpu_sc as plsc

_N = 10000
_E = 320000
_D = 128
_H = 4
_C = 32

_NC = 2
_NS = 16
_NW = _NC * _NS
_K = 128

_EPW = _E // _NW
_NCH = _EPW // _K
_TAIL = _EPW - _NCH * _K

_EPW2 = _E // _NS
_NCH2 = _EPW2 // _K
_TAIL2 = _EPW2 - _NCH2 * _K

_NP = 10240
_RPT = _NP // _NS

_f32 = jnp.float32


def _mm(a, b):
    return lax.dot_general(a, b, (((1,), (0,)), ((), ())),
                           preferred_element_type=_f32)


def _mmT(a, b):
    return lax.dot_general(a, b, (((1,), (1,)), ((), ())),
                           preferred_element_type=_f32)


def _sel16x128():
    rr = lax.broadcasted_iota(jnp.int32, (16, 128), 0)
    cc = lax.broadcasted_iota(jnp.int32, (16, 128), 1)
    return ((cc // _C) == rr).astype(_f32)


def _ag(attc):
    jr = lax.broadcasted_iota(jnp.int32, (128, 16), 0)
    hc = lax.broadcasted_iota(jnp.int32, (128, 16), 1)
    return ((jr // _C) == hc).astype(_f32) * attc


def _tc_xlr(x_ref, wl_ref, bl_ref, wr_ref, br_ref, xl_out, xr_out):
    xv = x_ref[...]
    xl_out[...] = _mmT(xv, wl_ref[...]) + bl_ref[...]
    xr_out[...] = _mmT(xv, wr_ref[...]) + br_ref[...]


def _tc_mid(gl_ref, gr_ref, ea_ref, we_ref, attc_ref, y2_out):
    gl = gl_ref[...]
    ea = ea_ref[...]
    ee = _mmT(ea, we_ref[...])
    m = gl + gr_ref[...] + ee
    m = jnp.maximum(m, 0.2 * m)
    l16 = _mm(m, _ag(attc_ref[...]))
    e16 = jnp.exp(l16)
    wex = _mm(e16, _sel16x128())
    r1 = lax.broadcasted_iota(jnp.int32, (16, 128), 0)
    c1 = lax.broadcasted_iota(jnp.int32, (16, 128), 1)
    Z1 = ((r1 == c1) & (r1 < 5)).astype(_f32)
    r2 = lax.broadcasted_iota(jnp.int32, (4, 128), 0)
    c2 = lax.broadcasted_iota(jnp.int32, (4, 128), 1)
    Z2 = (c2 == (r2 + 5)).astype(_f32)
    y2_out[0] = wex * gl
    y2_out[1] = _mm(e16, Z1) + _mm(ea, Z2)


def _tc_post(p2_ref, xl_ref, xr_ref, x_ref, we_ref, attc_ref, bias_ref,
             gw_ref, gb_ref, gms_ref, o_ref):
    p = p2_ref[...]
    acc = p[0]
    accz = p[1]
    ri = lax.broadcasted_iota(jnp.int32, (128, 128), 0)
    ci = lax.broadcasted_iota(jnp.int32, (128, 128), 1)
    denS = ((ci // _C) == ri).astype(_f32)
    r4 = lax.broadcasted_iota(jnp.int32, (128, 4), 0)
    c4 = lax.broadcasted_iota(jnp.int32, (128, 4), 1)
    D4 = (r4 == 4).astype(_f32)
    E4 = (r4 == (c4 + 5)).astype(_f32)
    den_part = _mm(accz, denS)
    deg4 = _mm(accz, D4)
    easum4 = _mm(accz, E4)
    la4 = easum4 / jnp.maximum(deg4, 1.0)
    ee = _mmT(la4, we_ref[...])
    xl = xl_ref[...]
    m = xl + xr_ref[...] + ee
    m = jnp.maximum(m, 0.2 * m)
    ls16 = _mm(m, _ag(attc_ref[...]))
    esl = jnp.exp(ls16)
    eslx = _mm(esl, _sel16x128())
    den = den_part + eslx
    out = (acc + eslx * xl) / den
    out = out + bias_ref[...] + x_ref[...]
    mean = jnp.mean(out, axis=0, keepdims=True)
    oc = out - mean * gms_ref[...]
    var = jnp.mean(oc * oc, axis=0, keepdims=True)
    outn = gw_ref[...] * oc / jnp.sqrt(var + 1e-5) + gb_ref[...]
    o_ref[...] = jnp.where(outn > 0.0, outn,
                           jnp.exp(jnp.minimum(outn, 0.0)) - 1.0)


def _sc_gather(xl, xr, src, dst, gl, gr,
               idxs, idxd, idxs_t, idxd_t, glv, grv, sem0, sem1):
    cid = lax.axis_index("c")
    sid = lax.axis_index("s")
    base = (sid * _NC + cid) * _EPW

    def body(i, carry):
        off = base + i * _K
        pltpu.sync_copy(src.at[pl.ds(off, _K)], idxs)
        pltpu.sync_copy(dst.at[pl.ds(off, _K)], idxd)
        c0 = pltpu.async_copy(xl.at[idxs], glv, sem0)
        c1 = pltpu.async_copy(xr.at[idxd], grv, sem1)
        c0.wait()
        c1.wait()
        pltpu.sync_copy(glv, gl.at[pl.ds(off, _K)])
        pltpu.sync_copy(grv, gr.at[pl.ds(off, _K)])
        return carry

    lax.fori_loop(0, _NCH, body, 0)
    offt = base + _NCH * _K
    pltpu.sync_copy(src.at[pl.ds(offt, _TAIL)], idxs_t)
    pltpu.sync_copy(dst.at[pl.ds(offt, _TAIL)], idxd_t)
    glt = glv.at[pl.ds(0, _TAIL)]
    grt = grv.at[pl.ds(0, _TAIL)]
    c0 = pltpu.async_copy(xl.at[idxs_t], glt, sem0)
    c1 = pltpu.async_copy(xr.at[idxd_t], grt, sem1)
    c0.wait()
    c1.wait()
    pltpu.sync_copy(glt, gl.at[pl.ds(offt, _TAIL)])
    pltpu.sync_copy(grt, gr.at[pl.ds(offt, _TAIL)])


def _sc_scatter(y2, dst, zeros128, out2, idx_v, idx_t, yv, yt, acc_sh):
    cid = lax.axis_index("c")
    sid = lax.axis_index("s")
    base = sid * _EPW2
    row0 = sid * _RPT
    pltpu.sync_copy(zeros128.at[pl.ds(row0, _RPT)],
                    acc_sh.at[pl.ds(row0, _RPT)])
    plsc.subcore_barrier()

    def body(i, carry):
        off = base + i * _K
        pltpu.sync_copy(dst.at[pl.ds(off, _K)], idx_v)
        pltpu.sync_copy(y2.at[cid, pl.ds(off, _K)], yv)
        pltpu.sync_copy(yv, acc_sh.at[idx_v], add=True)
        return carry

    lax.fori_loop(0, _NCH2, body, 0)
    offt = base + _NCH2 * _K
    pltpu.sync_copy(dst.at[pl.ds(offt, _TAIL2)], idx_t)
    pltpu.sync_copy(y2.at[cid, pl.ds(offt, _TAIL2)], yt)
    pltpu.sync_copy(yt, acc_sh.at[idx_t], add=True)
    plsc.subcore_barrier()
    pltpu.sync_copy(acc_sh.at[pl.ds(row0, _RPT)],
                    out2.at[cid, pl.ds(row0, _RPT)])


def _sc_mesh():
    return plsc.VectorSubcoreMesh(core_axis_name="c", subcore_axis_name="s")


_BE = 8000
_NBLK = _E // _BE


def kernel(x, edge_index, edge_attr, Wl, bl, Wr, br, We, att, bias, W_ea,
           b_ea, gn_weight, gn_bias, gn_mean_scale):
    src = edge_index[0]
    dst = edge_index[1]
    zeros128 = jnp.zeros((_NP, 128), _f32)
    attc = att.reshape(128, 1)
    bl2 = bl.reshape(1, 128)
    br2 = br.reshape(1, 128)
    bias2 = bias.reshape(1, 128)
    gw2 = gn_weight.reshape(1, 128)
    gb2 = gn_bias.reshape(1, 128)
    gms2 = gn_mean_scale.reshape(1, 128)

    x_l, x_r = pl.pallas_call(
        _tc_xlr,
        out_shape=[jax.ShapeDtypeStruct((_N, 128), _f32)] * 2,
    )(x, Wl, bl2, Wr, br2)

    gl, gr = pl.kernel(
        _sc_gather,
        out_type=(
            jax.ShapeDtypeStruct((_E, 128), _f32),
            jax.ShapeDtypeStruct((_E, 128), _f32),
        ),
        mesh=_sc_mesh(),
        scratch_types=[
            pltpu.VMEM((_K,), jnp.int32),
            pltpu.VMEM((_K,), jnp.int32),
            pltpu.VMEM((_TAIL,), jnp.int32),
            pltpu.VMEM((_TAIL,), jnp.int32),
            pltpu.VMEM((_K, 128), _f32),
            pltpu.VMEM((_K, 128), _f32),
            pltpu.SemaphoreType.DMA,
            pltpu.SemaphoreType.DMA,
        ],
    )(x_l, x_r, src, dst)

    y2 = pl.pallas_call(
        _tc_mid,
        grid=(_NBLK,),
        in_specs=[
            pl.BlockSpec((_BE, 128), lambda i: (i, 0)),
            pl.BlockSpec((_BE, 128), lambda i: (i, 0)),
            pl.BlockSpec((_BE, 4), lambda i: (i, 0)),
            pl.BlockSpec((128, 4), lambda i: (0, 0)),
            pl.BlockSpec((128, 1), lambda i: (0, 0)),
        ],
        out_specs=pl.BlockSpec((2, _BE, 128), lambda i: (0, i, 0)),
        out_shape=jax.ShapeDtypeStruct((2, _E, 128), _f32),
    )(gl, gr, edge_attr, We, attc)

    parts = pl.kernel(
        _sc_scatter,
        out_type=jax.ShapeDtypeStruct((2, _NP, 128), _f32),
        mesh=_sc_mesh(),
        scratch_types=[
            pltpu.VMEM((_K,), jnp.int32),
            pltpu.VMEM((_TAIL2,), jnp.int32),
            pltpu.VMEM((_K, 128), _f32),
            pltpu.VMEM((_TAIL2, 128), _f32),
            pltpu.VMEM_SHARED((_NP, 128), _f32),
        ],
    )(y2, dst, zeros128)
    parts = parts[:, :_N]

    out = pl.pallas_call(
        _tc_post,
        out_shape=jax.ShapeDtypeStruct((_N, 128), _f32),
    )(parts, x_l, x_r, x, We, attc, bias2, gw2, gb2, gms2)
    return out


# speedup vs baseline: 50.9548x; 1.2543x over previous
import jax
import jax.numpy as jnp
from jax import lax
from jax.experimental import pallas as pl
from jax.experimental.pallas import tpu as pltpu
from jax.experimental.pallas import tpu_sc as plsc

_N = 10000
_E = 320000
_D = 128
_H = 4
_C = 32

_NC = 2
_NS = 16
_NW = _NC * _NS
_K = 128

_EPW = _E // _NW
_NCH = _EPW // _K
_TAIL = _EPW - _NCH * _K

_EPW2 = _E // _NS
_NCH2 = _EPW2 // _K
_TAIL2 = _EPW2 - _NCH2 * _K

_NP = 10240
_RPT = _NP // _NS

_f32 = jnp.float32


def _mm(a, b):
    return lax.dot_general(a, b, (((1,), (0,)), ((), ())),
                           preferred_element_type=_f32)


def _mmT(a, b):
    return lax.dot_general(a, b, (((1,), (1,)), ((), ())),
                           preferred_element_type=_f32)


def _sel16x128():
    rr = lax.broadcasted_iota(jnp.int32, (16, 128), 0)
    cc = lax.broadcasted_iota(jnp.int32, (16, 128), 1)
    return ((cc // _C) == rr).astype(_f32)


def _ag(attc):
    jr = lax.broadcasted_iota(jnp.int32, (128, 16), 0)
    hc = lax.broadcasted_iota(jnp.int32, (128, 16), 1)
    return ((jr // _C) == hc).astype(_f32) * attc


def _tc_xlr(x_ref, wl_ref, bl_ref, wr_ref, br_ref, xl_out, xr_out):
    xv = x_ref[...]
    xl_out[...] = _mmT(xv, wl_ref[...]) + bl_ref[...]
    xr_out[...] = _mmT(xv, wr_ref[...]) + br_ref[...]


def _tc_mid(gl_ref, gr_ref, ea_ref, we_ref, attc_ref, y2_out):
    gl = gl_ref[...]
    ea = ea_ref[...]
    ee = _mmT(ea, we_ref[...])
    m = gl + gr_ref[...] + ee
    m = jnp.maximum(m, 0.2 * m)
    l16 = _mm(m, _ag(attc_ref[...]))
    e16 = jnp.exp(l16)
    wex = _mm(e16, _sel16x128())
    r1 = lax.broadcasted_iota(jnp.int32, (16, 128), 0)
    c1 = lax.broadcasted_iota(jnp.int32, (16, 128), 1)
    Z1 = ((r1 == c1) & (r1 < 5)).astype(_f32)
    r2 = lax.broadcasted_iota(jnp.int32, (4, 128), 0)
    c2 = lax.broadcasted_iota(jnp.int32, (4, 128), 1)
    Z2 = (c2 == (r2 + 5)).astype(_f32)
    y2_out[0] = wex * gl
    y2_out[1] = _mm(e16, Z1) + _mm(ea, Z2)


def _tc_post(p2_ref, xl_ref, xr_ref, x_ref, we_ref, attc_ref, bias_ref,
             gw_ref, gb_ref, gms_ref, o_ref):
    p = p2_ref[...]
    acc = p[0]
    accz = p[1]
    ri = lax.broadcasted_iota(jnp.int32, (128, 128), 0)
    ci = lax.broadcasted_iota(jnp.int32, (128, 128), 1)
    denS = ((ci // _C) == ri).astype(_f32)
    r4 = lax.broadcasted_iota(jnp.int32, (128, 4), 0)
    c4 = lax.broadcasted_iota(jnp.int32, (128, 4), 1)
    D4 = (r4 == 4).astype(_f32)
    E4 = (r4 == (c4 + 5)).astype(_f32)
    den_part = _mm(accz, denS)
    deg4 = _mm(accz, D4)
    easum4 = _mm(accz, E4)
    la4 = easum4 / jnp.maximum(deg4, 1.0)
    ee = _mmT(la4, we_ref[...])
    xl = xl_ref[...]
    m = xl + xr_ref[...] + ee
    m = jnp.maximum(m, 0.2 * m)
    ls16 = _mm(m, _ag(attc_ref[...]))
    esl = jnp.exp(ls16)
    eslx = _mm(esl, _sel16x128())
    den = den_part + eslx
    out = (acc + eslx * xl) / den
    out = out + bias_ref[...] + x_ref[...]
    mean = jnp.mean(out, axis=0, keepdims=True)
    oc = out - mean * gms_ref[...]
    var = jnp.mean(oc * oc, axis=0, keepdims=True)
    outn = gw_ref[...] * oc / jnp.sqrt(var + 1e-5) + gb_ref[...]
    o_ref[...] = jnp.where(outn > 0.0, outn,
                           jnp.exp(jnp.minimum(outn, 0.0)) - 1.0)


def _sc_gather(xl, xr, src, dst, gl, gr,
               idxs0, idxd0, idxs1, idxd1, glv0, grv0, glv1, grv1,
               idxs_t, idxd_t, glv_t, grv_t,
               sem_g0, sem_g1, sem_s0, sem_s1, sem_t):
    cid = lax.axis_index("c")
    sid = lax.axis_index("s")
    base = (sid * _NC + cid) * _EPW

    def load_idx(j, ix_s, ix_d):
        off = base + j * _K
        pltpu.sync_copy(src.at[pl.ds(off, _K)], ix_s)
        pltpu.sync_copy(dst.at[pl.ds(off, _K)], ix_d)

    def start_gather(ix_s, ix_d, bl, br_, sem):
        pltpu.async_copy(xl.at[ix_s], bl, sem)
        pltpu.async_copy(xr.at[ix_d], br_, sem)

    def wait_gather(bl, br_, sem):
        pltpu.make_async_copy(xl.at[pl.ds(0, _K)], bl, sem).wait()
        pltpu.make_async_copy(xr.at[pl.ds(0, _K)], br_, sem).wait()

    def start_store(j, bl, br_, sem):
        off = base + j * _K
        pltpu.async_copy(bl, gl.at[pl.ds(off, _K)], sem)
        pltpu.async_copy(br_, gr.at[pl.ds(off, _K)], sem)

    def wait_store(bl, br_, sem):
        pltpu.make_async_copy(bl, gl.at[pl.ds(0, _K)], sem).wait()
        pltpu.make_async_copy(br_, gr.at[pl.ds(0, _K)], sem).wait()

    load_idx(0, idxs0, idxd0)
    start_gather(idxs0, idxd0, glv0, grv0, sem_g0)
    load_idx(1, idxs1, idxd1)
    start_gather(idxs1, idxd1, glv1, grv1, sem_g1)
    wait_gather(glv0, grv0, sem_g0)
    start_store(0, glv0, grv0, sem_s0)
    wait_gather(glv1, grv1, sem_g1)
    start_store(1, glv1, grv1, sem_s1)

    def pair(p, carry):
        j0 = 2 * p
        wait_store(glv0, grv0, sem_s0)
        load_idx(j0, idxs0, idxd0)
        start_gather(idxs0, idxd0, glv0, grv0, sem_g0)
        wait_store(glv1, grv1, sem_s1)
        load_idx(j0 + 1, idxs1, idxd1)
        start_gather(idxs1, idxd1, glv1, grv1, sem_g1)
        wait_gather(glv0, grv0, sem_g0)
        start_store(j0, glv0, grv0, sem_s0)
        wait_gather(glv1, grv1, sem_g1)
        start_store(j0 + 1, glv1, grv1, sem_s1)
        return carry

    lax.fori_loop(1, _NCH // 2, pair, 0)

    offt = base + _NCH * _K
    pltpu.sync_copy(src.at[pl.ds(offt, _TAIL)], idxs_t)
    pltpu.sync_copy(dst.at[pl.ds(offt, _TAIL)], idxd_t)
    c0 = pltpu.async_copy(xl.at[idxs_t], glv_t, sem_t)
    c1 = pltpu.async_copy(xr.at[idxd_t], grv_t, sem_t)
    c0.wait()
    c1.wait()
    pltpu.sync_copy(glv_t, gl.at[pl.ds(offt, _TAIL)])
    pltpu.sync_copy(grv_t, gr.at[pl.ds(offt, _TAIL)])
    wait_store(glv0, grv0, sem_s0)
    wait_store(glv1, grv1, sem_s1)


def _sc_scatter(y2, dst, zeros128, out2,
                idx0, idx1, idx_t, yv0, yv1, yt,
                sem_y0, sem_y1, sem_c0, sem_c1, acc_sh):
    cid = lax.axis_index("c")
    sid = lax.axis_index("s")
    base = sid * _EPW2
    row0 = sid * _RPT
    pltpu.sync_copy(zeros128.at[pl.ds(row0, _RPT)],
                    acc_sh.at[pl.ds(row0, _RPT)])
    plsc.subcore_barrier()

    def load(j, ix, yv, sem):
        off = base + j * _K
        pltpu.sync_copy(dst.at[pl.ds(off, _K)], ix)
        pltpu.async_copy(y2.at[cid, pl.ds(off, _K)], yv, sem)

    def wait_load(yv, sem):
        pltpu.make_async_copy(y2.at[cid, pl.ds(0, _K)], yv, sem).wait()

    def start_scatter(ix, yv, sem):
        pltpu.async_copy(yv, acc_sh.at[ix], sem, add=True)

    def wait_scatter(yv, sem):
        pltpu.make_async_copy(yv, acc_sh.at[pl.ds(0, _K)], sem).wait()

    load(0, idx0, yv0, sem_y0)
    load(1, idx1, yv1, sem_y1)
    wait_load(yv0, sem_y0)
    start_scatter(idx0, yv0, sem_c0)
    wait_load(yv1, sem_y1)
    start_scatter(idx1, yv1, sem_c1)

    def pair(p, carry):
        j0 = 2 * p
        wait_scatter(yv0, sem_c0)
        load(j0, idx0, yv0, sem_y0)
        wait_scatter(yv1, sem_c1)
        load(j0 + 1, idx1, yv1, sem_y1)
        wait_load(yv0, sem_y0)
        start_scatter(idx0, yv0, sem_c0)
        wait_load(yv1, sem_y1)
        start_scatter(idx1, yv1, sem_c1)
        return carry

    lax.fori_loop(1, _NCH2 // 2, pair, 0)
    wait_scatter(yv0, sem_c0)
    wait_scatter(yv1, sem_c1)

    offt = base + _NCH2 * _K
    pltpu.sync_copy(dst.at[pl.ds(offt, _TAIL2)], idx_t)
    pltpu.sync_copy(y2.at[cid, pl.ds(offt, _TAIL2)], yt)
    pltpu.sync_copy(yt, acc_sh.at[idx_t], add=True)
    plsc.subcore_barrier()
    pltpu.sync_copy(acc_sh.at[pl.ds(row0, _RPT)],
                    out2.at[cid, pl.ds(row0, _RPT)])


def _sc_mesh():
    return plsc.VectorSubcoreMesh(core_axis_name="c", subcore_axis_name="s")


_BE = 8000
_NBLK = _E // _BE


def kernel(x, edge_index, edge_attr, Wl, bl, Wr, br, We, att, bias, W_ea,
           b_ea, gn_weight, gn_bias, gn_mean_scale):
    src = edge_index[0]
    dst = edge_index[1]
    zeros128 = jnp.zeros((_NP, 128), _f32)
    attc = att.reshape(128, 1)
    bl2 = bl.reshape(1, 128)
    br2 = br.reshape(1, 128)
    bias2 = bias.reshape(1, 128)
    gw2 = gn_weight.reshape(1, 128)
    gb2 = gn_bias.reshape(1, 128)
    gms2 = gn_mean_scale.reshape(1, 128)

    x_l, x_r = pl.pallas_call(
        _tc_xlr,
        out_shape=[jax.ShapeDtypeStruct((_N, 128), _f32)] * 2,
    )(x, Wl, bl2, Wr, br2)

    gl, gr = pl.kernel(
        _sc_gather,
        out_type=(
            jax.ShapeDtypeStruct((_E, 128), _f32),
            jax.ShapeDtypeStruct((_E, 128), _f32),
        ),
        mesh=_sc_mesh(),
        scratch_types=[
            pltpu.VMEM((_K,), jnp.int32),
            pltpu.VMEM((_K,), jnp.int32),
            pltpu.VMEM((_K,), jnp.int32),
            pltpu.VMEM((_K,), jnp.int32),
            pltpu.VMEM((_K, 128), _f32),
            pltpu.VMEM((_K, 128), _f32),
            pltpu.VMEM((_K, 128), _f32),
            pltpu.VMEM((_K, 128), _f32),
            pltpu.VMEM((_TAIL,), jnp.int32),
            pltpu.VMEM((_TAIL,), jnp.int32),
            pltpu.VMEM((_TAIL, 128), _f32),
            pltpu.VMEM((_TAIL, 128), _f32),
            pltpu.SemaphoreType.DMA,
            pltpu.SemaphoreType.DMA,
            pltpu.SemaphoreType.DMA,
            pltpu.SemaphoreType.DMA,
            pltpu.SemaphoreType.DMA,
        ],
    )(x_l, x_r, src, dst)

    y2 = pl.pallas_call(
        _tc_mid,
        grid=(_NBLK,),
        in_specs=[
            pl.BlockSpec((_BE, 128), lambda i: (i, 0)),
            pl.BlockSpec((_BE, 128), lambda i: (i, 0)),
            pl.BlockSpec((_BE, 4), lambda i: (i, 0)),
            pl.BlockSpec((128, 4), lambda i: (0, 0)),
            pl.BlockSpec((128, 1), lambda i: (0, 0)),
        ],
        out_specs=pl.BlockSpec((2, _BE, 128), lambda i: (0, i, 0)),
        out_shape=jax.ShapeDtypeStruct((2, _E, 128), _f32),
    )(gl, gr, edge_attr, We, attc)

    parts = pl.kernel(
        _sc_scatter,
        out_type=jax.ShapeDtypeStruct((2, _NP, 128), _f32),
        mesh=_sc_mesh(),
        scratch_types=[
            pltpu.VMEM((_K,), jnp.int32),
            pltpu.VMEM((_K,), jnp.int32),
            pltpu.VMEM((_TAIL2,), jnp.int32),
            pltpu.VMEM((_K, 128), _f32),
            pltpu.VMEM((_K, 128), _f32),
            pltpu.VMEM((_TAIL2, 128), _f32),
            pltpu.SemaphoreType.DMA,
            pltpu.SemaphoreType.DMA,
            pltpu.SemaphoreType.DMA,
            pltpu.SemaphoreType.DMA,
            pltpu.VMEM_SHARED((_NP, 128), _f32),
        ],
    )(y2, dst, zeros128)
    parts = parts[:, :_N]

    out = pl.pallas_call(
        _tc_post,
        out_shape=jax.ShapeDtypeStruct((_N, 128), _f32),
    )(parts, x_l, x_r, x, We, attc, bias2, gw2, gb2, gms2)
    return out


# speedup vs baseline: 51.9095x; 1.0187x over previous
import jax
import jax.numpy as jnp
from jax import lax
from jax.experimental import pallas as pl
from jax.experimental.pallas import tpu as pltpu
from jax.experimental.pallas import tpu_sc as plsc

_N = 10000
_E = 320000
_D = 128
_H = 4
_C = 32

_NC = 2
_NS = 16
_NW = _NC * _NS
_K = 128

_NCH = 78
_EPW = _NCH * _K
_XBASE = _EPW * _NW

_EPW2 = _E // _NS
_NCH2 = _EPW2 // _K
_TAIL2 = _EPW2 - _NCH2 * _K

_NP = 10240
_RPT = _NP // _NS

_f32 = jnp.float32


def _mm(a, b):
    return lax.dot_general(a, b, (((1,), (0,)), ((), ())),
                           preferred_element_type=_f32)


def _mmT(a, b):
    return lax.dot_general(a, b, (((1,), (1,)), ((), ())),
                           preferred_element_type=_f32)


def _sel16x128():
    rr = lax.broadcasted_iota(jnp.int32, (16, 128), 0)
    cc = lax.broadcasted_iota(jnp.int32, (16, 128), 1)
    return ((cc // _C) == rr).astype(_f32)


def _ag(attc):
    jr = lax.broadcasted_iota(jnp.int32, (128, 16), 0)
    hc = lax.broadcasted_iota(jnp.int32, (128, 16), 1)
    return ((jr // _C) == hc).astype(_f32) * attc


def _tc_xlr(x_ref, wl_ref, bl_ref, wr_ref, br_ref, xl_out, xr_out):
    xv = x_ref[...]
    xl_out[...] = _mmT(xv, wl_ref[...]) + bl_ref[...]
    xr_out[...] = _mmT(xv, wr_ref[...]) + br_ref[...]


def _tc_mid(gl_ref, gr_ref, ea_ref, we_ref, attc_ref, y2_out):
    gl = gl_ref[...]
    ea = ea_ref[...]
    ee = _mmT(ea, we_ref[...])
    m = gl + gr_ref[...] + ee
    m = jnp.maximum(m, 0.2 * m)
    l16 = _mm(m, _ag(attc_ref[...]))
    e16 = jnp.exp(l16)
    wex = _mm(e16, _sel16x128())
    r1 = lax.broadcasted_iota(jnp.int32, (16, 128), 0)
    c1 = lax.broadcasted_iota(jnp.int32, (16, 128), 1)
    Z1 = ((r1 == c1) & (r1 < 5)).astype(_f32)
    r2 = lax.broadcasted_iota(jnp.int32, (4, 128), 0)
    c2 = lax.broadcasted_iota(jnp.int32, (4, 128), 1)
    Z2 = (c2 == (r2 + 5)).astype(_f32)
    y2_out[0] = wex * gl
    y2_out[1] = _mm(e16, Z1) + _mm(ea, Z2)


def _tc_post(p2_ref, xl_ref, xr_ref, x_ref, we_ref, attc_ref, bias_ref,
             gw_ref, gb_ref, gms_ref, o_ref):
    p = p2_ref[...]
    acc = p[0]
    accz = p[1]
    ri = lax.broadcasted_iota(jnp.int32, (128, 128), 0)
    ci = lax.broadcasted_iota(jnp.int32, (128, 128), 1)
    denS = ((ci // _C) == ri).astype(_f32)
    r4 = lax.broadcasted_iota(jnp.int32, (128, 4), 0)
    c4 = lax.broadcasted_iota(jnp.int32, (128, 4), 1)
    D4 = (r4 == 4).astype(_f32)
    E4 = (r4 == (c4 + 5)).astype(_f32)
    den_part = _mm(accz, denS)
    deg4 = _mm(accz, D4)
    easum4 = _mm(accz, E4)
    la4 = easum4 / jnp.maximum(deg4, 1.0)
    ee = _mmT(la4, we_ref[...])
    xl = xl_ref[...]
    m = xl + xr_ref[...] + ee
    m = jnp.maximum(m, 0.2 * m)
    ls16 = _mm(m, _ag(attc_ref[...]))
    esl = jnp.exp(ls16)
    eslx = _mm(esl, _sel16x128())
    den = den_part + eslx
    out = (acc + eslx * xl) / den
    out = out + bias_ref[...] + x_ref[...]
    mean = jnp.mean(out, axis=0, keepdims=True)
    oc = out - mean * gms_ref[...]
    var = jnp.mean(oc * oc, axis=0, keepdims=True)
    outn = gw_ref[...] * oc / jnp.sqrt(var + 1e-5) + gb_ref[...]
    o_ref[...] = jnp.where(outn > 0.0, outn,
                           jnp.exp(jnp.minimum(outn, 0.0)) - 1.0)


_NB = 3
_NTRI = _NCH // _NB


def _sc_gather(xl, xr, eidx, gl, gr,
               ix0, ix1, ix2, glv0, grv0, glv1, grv1, glv2, grv2,
               sg0, sg1, sg2, ss0, ss1, ss2):
    cid = lax.axis_index("c")
    sid = lax.axis_index("s")
    wid = sid * _NC + cid
    base = wid * _EPW
    bufs = ((ix0, glv0, grv0, sg0, ss0),
            (ix1, glv1, grv1, sg1, ss1),
            (ix2, glv2, grv2, sg2, ss2))

    def load_idx(j, ix):
        off = base + j * _K
        pltpu.sync_copy(eidx.at[:, pl.ds(off, _K)], ix)

    def start_gather(ix, bl, br_, sem):
        pltpu.async_copy(xl.at[ix.at[0]], bl, sem)
        pltpu.async_copy(xr.at[ix.at[1]], br_, sem)

    def wait_gather(bl, br_, sem):
        pltpu.make_async_copy(xl.at[pl.ds(0, _K)], bl, sem).wait()
        pltpu.make_async_copy(xr.at[pl.ds(0, _K)], br_, sem).wait()

    def start_store(j, bl, br_, sem):
        off = base + j * _K
        pltpu.async_copy(bl, gl.at[pl.ds(off, _K)], sem)
        pltpu.async_copy(br_, gr.at[pl.ds(off, _K)], sem)

    def wait_store(bl, br_, sem):
        pltpu.make_async_copy(bl, gl.at[pl.ds(0, _K)], sem).wait()
        pltpu.make_async_copy(br_, gr.at[pl.ds(0, _K)], sem).wait()

    for b, (ix, bl, br_, sg, ss) in enumerate(bufs):
        load_idx(b, ix)
        start_gather(ix, bl, br_, sg)
    for b, (ix, bl, br_, sg, ss) in enumerate(bufs):
        wait_gather(bl, br_, sg)
        start_store(b, bl, br_, ss)

    def rot(t, carry):
        j0 = _NB * t
        for b, (ix, bl, br_, sg, ss) in enumerate(bufs):
            wait_store(bl, br_, ss)
            load_idx(j0 + b, ix)
            start_gather(ix, bl, br_, sg)
        for b, (ix, bl, br_, sg, ss) in enumerate(bufs):
            wait_gather(bl, br_, sg)
            start_store(j0 + b, bl, br_, ss)
        return carry

    lax.fori_loop(1, _NTRI, rot, 0)

    for ix, bl, br_, sg, ss in bufs:
        wait_store(bl, br_, ss)

    @pl.when(wid < 4)
    def _extra():
        offx = _XBASE + wid * _K
        pltpu.sync_copy(eidx.at[:, pl.ds(offx, _K)], ix0)
        c0 = pltpu.async_copy(xl.at[ix0.at[0]], glv0, sg0)
        c1 = pltpu.async_copy(xr.at[ix0.at[1]], grv0, sg0)
        c0.wait()
        c1.wait()
        pltpu.sync_copy(glv0, gl.at[pl.ds(offx, _K)])
        pltpu.sync_copy(grv0, gr.at[pl.ds(offx, _K)])


def _sc_scatter(y2, dst, zeros128, out2,
                idx0, idx1, idx_t, yv0, yv1, yt,
                sy0, sy1, sc0, sc1, acc_sh):
    cid = lax.axis_index("c")
    sid = lax.axis_index("s")
    base = sid * _EPW2
    row0 = sid * _RPT
    pltpu.sync_copy(zeros128.at[pl.ds(row0, _RPT)],
                    acc_sh.at[pl.ds(row0, _RPT)])
    plsc.subcore_barrier()
    bufs = ((idx0, yv0, sy0, sc0), (idx1, yv1, sy1, sc1))

    def load(j, ix, yv, sem):
        off = base + j * _K
        pltpu.sync_copy(dst.at[pl.ds(off, _K)], ix)
        pltpu.async_copy(y2.at[cid, pl.ds(off, _K)], yv, sem)

    def wait_load(yv, sem):
        pltpu.make_async_copy(y2.at[cid, pl.ds(0, _K)], yv, sem).wait()

    def start_scatter(ix, yv, sem):
        pltpu.async_copy(yv, acc_sh.at[ix], sem, add=True)

    def wait_scatter(yv, sem):
        pltpu.make_async_copy(yv, acc_sh.at[pl.ds(0, _K)], sem).wait()

    for b, (ix, yv, sy, sc) in enumerate(bufs):
        load(b, ix, yv, sy)
    for b, (ix, yv, sy, sc) in enumerate(bufs):
        wait_load(yv, sy)
        start_scatter(ix, yv, sc)

    def rot(t, carry):
        j0 = 2 * t
        for b, (ix, yv, sy, sc) in enumerate(bufs):
            wait_scatter(yv, sc)
            load(j0 + b, ix, yv, sy)
        for b, (ix, yv, sy, sc) in enumerate(bufs):
            wait_load(yv, sy)
            start_scatter(ix, yv, sc)
        return carry

    lax.fori_loop(1, _NCH2 // 2, rot, 0)
    for ix, yv, sy, sc in bufs:
        wait_scatter(yv, sc)

    offt = base + _NCH2 * _K
    pltpu.sync_copy(dst.at[pl.ds(offt, _TAIL2)], idx_t)
    pltpu.sync_copy(y2.at[cid, pl.ds(offt, _TAIL2)], yt)
    pltpu.sync_copy(yt, acc_sh.at[idx_t], add=True)
    plsc.subcore_barrier()
    pltpu.sync_copy(acc_sh.at[pl.ds(row0, _RPT)],
                    out2.at[cid, pl.ds(row0, _RPT)])


def _sc_mesh():
    return plsc.VectorSubcoreMesh(core_axis_name="c", subcore_axis_name="s")


_BE = 8000
_NBLK = _E // _BE


def kernel(x, edge_index, edge_attr, Wl, bl, Wr, br, We, att, bias, W_ea,
           b_ea, gn_weight, gn_bias, gn_mean_scale):
    src = edge_index[0]
    dst = edge_index[1]
    zeros128 = jnp.zeros((_NP, 128), _f32)
    attc = att.reshape(128, 1)
    bl2 = bl.reshape(1, 128)
    br2 = br.reshape(1, 128)
    bias2 = bias.reshape(1, 128)
    gw2 = gn_weight.reshape(1, 128)
    gb2 = gn_bias.reshape(1, 128)
    gms2 = gn_mean_scale.reshape(1, 128)

    x_l, x_r = pl.pallas_call(
        _tc_xlr,
        out_shape=[jax.ShapeDtypeStruct((_N, 128), _f32)] * 2,
    )(x, Wl, bl2, Wr, br2)

    gl, gr = pl.kernel(
        _sc_gather,
        out_type=(
            jax.ShapeDtypeStruct((_E, 128), _f32),
            jax.ShapeDtypeStruct((_E, 128), _f32),
        ),
        mesh=_sc_mesh(),
        scratch_types=[
            pltpu.VMEM((2, _K), jnp.int32),
            pltpu.VMEM((2, _K), jnp.int32),
            pltpu.VMEM((2, _K), jnp.int32),
            pltpu.VMEM((_K, 128), _f32),
            pltpu.VMEM((_K, 128), _f32),
            pltpu.VMEM((_K, 128), _f32),
            pltpu.VMEM((_K, 128), _f32),
            pltpu.VMEM((_K, 128), _f32),
            pltpu.VMEM((_K, 128), _f32),
            pltpu.SemaphoreType.DMA,
            pltpu.SemaphoreType.DMA,
            pltpu.SemaphoreType.DMA,
            pltpu.SemaphoreType.DMA,
            pltpu.SemaphoreType.DMA,
            pltpu.SemaphoreType.DMA,
        ],
    )(x_l, x_r, edge_index)

    y2 = pl.pallas_call(
        _tc_mid,
        grid=(_NBLK,),
        in_specs=[
            pl.BlockSpec((_BE, 128), lambda i: (i, 0)),
            pl.BlockSpec((_BE, 128), lambda i: (i, 0)),
            pl.BlockSpec((_BE, 4), lambda i: (i, 0)),
            pl.BlockSpec((128, 4), lambda i: (0, 0)),
            pl.BlockSpec((128, 1), lambda i: (0, 0)),
        ],
        out_specs=pl.BlockSpec((2, _BE, 128), lambda i: (0, i, 0)),
        out_shape=jax.ShapeDtypeStruct((2, _E, 128), _f32),
    )(gl, gr, edge_attr, We, attc)

    parts = pl.kernel(
        _sc_scatter,
        out_type=jax.ShapeDtypeStruct((2, _NP, 128), _f32),
        mesh=_sc_mesh(),
        scratch_types=[
            pltpu.VMEM((_K,), jnp.int32),
            pltpu.VMEM((_K,), jnp.int32),
            pltpu.VMEM((_TAIL2,), jnp.int32),
            pltpu.VMEM((_K, 128), _f32),
            pltpu.VMEM((_K, 128), _f32),
            pltpu.VMEM((_TAIL2, 128), _f32),
            pltpu.SemaphoreType.DMA,
            pltpu.SemaphoreType.DMA,
            pltpu.SemaphoreType.DMA,
            pltpu.SemaphoreType.DMA,
            pltpu.VMEM_SHARED((_NP, 128), _f32),
        ],
    )(y2, dst, zeros128)
    parts = parts[:, :_N]

    out = pl.pallas_call(
        _tc_post,
        out_shape=jax.ShapeDtypeStruct((_N, 128), _f32),
    )(parts, x_l, x_r, x, We, attc, bias2, gw2, gb2, gms2)
    return out


# speedup vs baseline: 54.5503x; 1.0509x over previous
import jax
import jax.numpy as jnp
from jax import lax
from jax.experimental import pallas as pl
from jax.experimental.pallas import tpu as pltpu
from jax.experimental.pallas import tpu_sc as plsc

_N = 10000
_E = 320000
_D = 128
_H = 4
_C = 32

_NC = 2
_NS = 16
_NW = _NC * _NS
_K = 128

_NCH = 78
_EPW = _NCH * _K
_XBASE = _EPW * _NW

_EPW2 = _E // _NS
_NCH2 = _EPW2 // _K
_TAIL2 = _EPW2 - _NCH2 * _K

_NP = 10240
_RPT = _NP // _NS

_f32 = jnp.float32


def _mm(a, b):
    return lax.dot_general(a, b, (((1,), (0,)), ((), ())),
                           preferred_element_type=_f32)


def _mmT(a, b):
    return lax.dot_general(a, b, (((1,), (1,)), ((), ())),
                           preferred_element_type=_f32)


def _sel16x128():
    rr = lax.broadcasted_iota(jnp.int32, (16, 128), 0)
    cc = lax.broadcasted_iota(jnp.int32, (16, 128), 1)
    return ((cc // _C) == rr).astype(_f32)


def _ag(attc):
    jr = lax.broadcasted_iota(jnp.int32, (128, 16), 0)
    hc = lax.broadcasted_iota(jnp.int32, (128, 16), 1)
    return ((jr // _C) == hc).astype(_f32) * attc


def _tc_xlr(x_ref, wl_ref, bl_ref, wr_ref, br_ref, xl_out, xr_out):
    xv = x_ref[...]
    xl_out[...] = _mmT(xv, wl_ref[...]) + bl_ref[...]
    xr_out[...] = _mmT(xv, wr_ref[...]) + br_ref[...]


def _tc_mid(gl_ref, gr_ref, ea_ref, we_ref, attc_ref, y2_out):
    gl = gl_ref[...]
    ea = ea_ref[...]
    ee = _mmT(ea, we_ref[...])
    m = gl + gr_ref[...] + ee
    m = jnp.maximum(m, 0.2 * m)
    l16 = _mm(m, _ag(attc_ref[...]))
    e16 = jnp.exp(l16)
    wex = _mm(e16, _sel16x128())
    r1 = lax.broadcasted_iota(jnp.int32, (16, 128), 0)
    c1 = lax.broadcasted_iota(jnp.int32, (16, 128), 1)
    Z1 = ((r1 == c1) & (r1 < 5)).astype(_f32)
    r2 = lax.broadcasted_iota(jnp.int32, (4, 128), 0)
    c2 = lax.broadcasted_iota(jnp.int32, (4, 128), 1)
    Z2 = (c2 == (r2 + 5)).astype(_f32)
    y2_out[0] = wex * gl
    y2_out[1] = _mm(e16, Z1) + _mm(ea, Z2)


def _tc_post(pa_ref, pb_ref, xl_ref, xr_ref, x_ref, we_ref, attc_ref,
             bias_ref, gw_ref, gb_ref, gms_ref, o_ref):
    p = pa_ref[...] + pb_ref[...]
    acc = p[0]
    accz = p[1]
    ri = lax.broadcasted_iota(jnp.int32, (128, 128), 0)
    ci = lax.broadcasted_iota(jnp.int32, (128, 128), 1)
    denS = ((ci // _C) == ri).astype(_f32)
    r4 = lax.broadcasted_iota(jnp.int32, (128, 4), 0)
    c4 = lax.broadcasted_iota(jnp.int32, (128, 4), 1)
    D4 = (r4 == 4).astype(_f32)
    E4 = (r4 == (c4 + 5)).astype(_f32)
    den_part = _mm(accz, denS)
    deg4 = _mm(accz, D4)
    easum4 = _mm(accz, E4)
    la4 = easum4 / jnp.maximum(deg4, 1.0)
    ee = _mmT(la4, we_ref[...])
    xl = xl_ref[...]
    m = xl + xr_ref[...] + ee
    m = jnp.maximum(m, 0.2 * m)
    ls16 = _mm(m, _ag(attc_ref[...]))
    esl = jnp.exp(ls16)
    eslx = _mm(esl, _sel16x128())
    den = den_part + eslx
    out = (acc + eslx * xl) / den
    out = out + bias_ref[...] + x_ref[...]
    mean = jnp.mean(out, axis=0, keepdims=True)
    oc = out - mean * gms_ref[...]
    var = jnp.mean(oc * oc, axis=0, keepdims=True)
    outn = gw_ref[...] * oc / jnp.sqrt(var + 1e-5) + gb_ref[...]
    o_ref[...] = jnp.where(outn > 0.0, outn,
                           jnp.exp(jnp.minimum(outn, 0.0)) - 1.0)


_NB = 3
_NTRI = _NCH // _NB


def _make_gather(nch, nxtra, xbase):
    ntri = nch // _NB
    epw = nch * _K

    def gather(xl, xr, eidx, gl, gr,
               ix0, ix1, ix2, glv0, grv0, glv1, grv1, glv2, grv2,
               sg0, sg1, sg2, ss0, ss1, ss2):
        cid = lax.axis_index("c")
        sid = lax.axis_index("s")
        wid = sid * _NC + cid
        base = wid * epw
        bufs = ((ix0, glv0, grv0, sg0, ss0),
                (ix1, glv1, grv1, sg1, ss1),
                (ix2, glv2, grv2, sg2, ss2))

        def load_idx(j, ix):
            off = base + j * _K
            pltpu.sync_copy(eidx.at[:, pl.ds(off, _K)], ix)

        def start_gather(ix, bl, br_, sem):
            pltpu.async_copy(xl.at[ix.at[0]], bl, sem)
            pltpu.async_copy(xr.at[ix.at[1]], br_, sem)

        def wait_gather(bl, br_, sem):
            pltpu.make_async_copy(xl.at[pl.ds(0, _K)], bl, sem).wait()
            pltpu.make_async_copy(xr.at[pl.ds(0, _K)], br_, sem).wait()

        def start_store(j, bl, br_, sem):
            off = base + j * _K
            pltpu.async_copy(bl, gl.at[pl.ds(off, _K)], sem)
            pltpu.async_copy(br_, gr.at[pl.ds(off, _K)], sem)

        def wait_store(bl, br_, sem):
            pltpu.make_async_copy(bl, gl.at[pl.ds(0, _K)], sem).wait()
            pltpu.make_async_copy(br_, gr.at[pl.ds(0, _K)], sem).wait()

        for b, (ix, bl, br_, sg, ss) in enumerate(bufs):
            load_idx(b, ix)
            start_gather(ix, bl, br_, sg)
        for b, (ix, bl, br_, sg, ss) in enumerate(bufs):
            wait_gather(bl, br_, sg)
            start_store(b, bl, br_, ss)

        def rot(t, carry):
            j0 = _NB * t
            for b, (ix, bl, br_, sg, ss) in enumerate(bufs):
                wait_store(bl, br_, ss)
                load_idx(j0 + b, ix)
                start_gather(ix, bl, br_, sg)
            for b, (ix, bl, br_, sg, ss) in enumerate(bufs):
                wait_gather(bl, br_, sg)
                start_store(j0 + b, bl, br_, ss)
            return carry

        lax.fori_loop(1, ntri, rot, 0)
        for ix, bl, br_, sg, ss in bufs:
            wait_store(bl, br_, ss)

        @pl.when(wid < nxtra)
        def _extra():
            offx = xbase + wid * _K
            pltpu.sync_copy(eidx.at[:, pl.ds(offx, _K)], ix0)
            c0 = pltpu.async_copy(xl.at[ix0.at[0]], glv0, sg0)
            c1 = pltpu.async_copy(xr.at[ix0.at[1]], grv0, sg0)
            c0.wait()
            c1.wait()
            pltpu.sync_copy(glv0, gl.at[pl.ds(offx, _K)])
            pltpu.sync_copy(grv0, gr.at[pl.ds(offx, _K)])

    return gather


def _make_scatter(nch2, tail2):
    epw2 = nch2 * _K + tail2

    def scatter(y2, dst, zeros128, out2,
                idx0, idx1, idx_t, yv0, yv1, yt,
                sy0, sy1, sc0, sc1, acc_sh):
        cid = lax.axis_index("c")
        sid = lax.axis_index("s")
        base = sid * epw2
        row0 = sid * _RPT
        pltpu.sync_copy(zeros128.at[pl.ds(row0, _RPT)],
                        acc_sh.at[pl.ds(row0, _RPT)])
        plsc.subcore_barrier()
        bufs = ((idx0, yv0, sy0, sc0), (idx1, yv1, sy1, sc1))

        def load(j, ix, yv, sem):
            off = base + j * _K
            pltpu.sync_copy(dst.at[pl.ds(off, _K)], ix)
            pltpu.async_copy(y2.at[cid, pl.ds(off, _K)], yv, sem)

        def wait_load(yv, sem):
            pltpu.make_async_copy(y2.at[cid, pl.ds(0, _K)], yv, sem).wait()

        def start_scatter(ix, yv, sem):
            pltpu.async_copy(yv, acc_sh.at[ix], sem, add=True)

        def wait_scatter(yv, sem):
            pltpu.make_async_copy(yv, acc_sh.at[pl.ds(0, _K)], sem).wait()

        for b, (ix, yv, sy, sc) in enumerate(bufs):
            load(b, ix, yv, sy)
        for b, (ix, yv, sy, sc) in enumerate(bufs):
            wait_load(yv, sy)
            start_scatter(ix, yv, sc)

        def rot(t, carry):
            j0 = 2 * t
            for b, (ix, yv, sy, sc) in enumerate(bufs):
                wait_scatter(yv, sc)
                load(j0 + b, ix, yv, sy)
            for b, (ix, yv, sy, sc) in enumerate(bufs):
                wait_load(yv, sy)
                start_scatter(ix, yv, sc)
            return carry

        lax.fori_loop(1, nch2 // 2, rot, 0)
        for ix, yv, sy, sc in bufs:
            wait_scatter(yv, sc)

        offt = base + nch2 * _K
        pltpu.sync_copy(dst.at[pl.ds(offt, tail2)], idx_t)
        pltpu.sync_copy(y2.at[cid, pl.ds(offt, tail2)], yt)
        pltpu.sync_copy(yt, acc_sh.at[idx_t], add=True)
        plsc.subcore_barrier()
        pltpu.sync_copy(acc_sh.at[pl.ds(row0, _RPT)],
                        out2.at[cid, pl.ds(row0, _RPT)])

    return scatter


def _sc_mesh():
    return plsc.VectorSubcoreMesh(core_axis_name="c", subcore_axis_name="s")


_BE = 8000
_EH = _E // 2
_NBLKH = _EH // _BE
_NCHH = 39
_XTRA = 2
_XBASEH = _NCHH * _K * _NW
_NCH2H = 78
_TAIL2H = 16

_gather_half = _make_gather(_NCHH, _XTRA, _XBASEH)
_scatter_half = _make_scatter(_NCH2H, _TAIL2H)

_GATHER_SCRATCH = [
    pltpu.VMEM((2, _K), jnp.int32),
    pltpu.VMEM((2, _K), jnp.int32),
    pltpu.VMEM((2, _K), jnp.int32),
    pltpu.VMEM((_K, 128), _f32),
    pltpu.VMEM((_K, 128), _f32),
    pltpu.VMEM((_K, 128), _f32),
    pltpu.VMEM((_K, 128), _f32),
    pltpu.VMEM((_K, 128), _f32),
    pltpu.VMEM((_K, 128), _f32),
    pltpu.SemaphoreType.DMA,
    pltpu.SemaphoreType.DMA,
    pltpu.SemaphoreType.DMA,
    pltpu.SemaphoreType.DMA,
    pltpu.SemaphoreType.DMA,
    pltpu.SemaphoreType.DMA,
]

_SCATTER_SCRATCH = [
    pltpu.VMEM((_K,), jnp.int32),
    pltpu.VMEM((_K,), jnp.int32),
    pltpu.VMEM((_TAIL2H,), jnp.int32),
    pltpu.VMEM((_K, 128), _f32),
    pltpu.VMEM((_K, 128), _f32),
    pltpu.VMEM((_TAIL2H, 128), _f32),
    pltpu.SemaphoreType.DMA,
    pltpu.SemaphoreType.DMA,
    pltpu.SemaphoreType.DMA,
    pltpu.SemaphoreType.DMA,
    pltpu.VMEM_SHARED((_NP, 128), _f32),
]


def _run_gather(x_l, x_r, eidx_h):
    return pl.kernel(
        _gather_half,
        out_type=(
            jax.ShapeDtypeStruct((_EH, 128), _f32),
            jax.ShapeDtypeStruct((_EH, 128), _f32),
        ),
        mesh=_sc_mesh(),
        scratch_types=_GATHER_SCRATCH,
    )(x_l, x_r, eidx_h)


def _run_mid(gl, gr, ea_h, We, attc):
    return pl.pallas_call(
        _tc_mid,
        grid=(_NBLKH,),
        in_specs=[
            pl.BlockSpec((_BE, 128), lambda i: (i, 0)),
            pl.BlockSpec((_BE, 128), lambda i: (i, 0)),
            pl.BlockSpec((_BE, 4), lambda i: (i, 0)),
            pl.BlockSpec((128, 4), lambda i: (0, 0)),
            pl.BlockSpec((128, 1), lambda i: (0, 0)),
        ],
        out_specs=pl.BlockSpec((2, _BE, 128), lambda i: (0, i, 0)),
        out_shape=jax.ShapeDtypeStruct((2, _EH, 128), _f32),
    )(gl, gr, ea_h, We, attc)


def _run_scatter(y2, dst_h, zeros128):
    return pl.kernel(
        _scatter_half,
        out_type=jax.ShapeDtypeStruct((2, _NP, 128), _f32),
        mesh=_sc_mesh(),
        scratch_types=_SCATTER_SCRATCH,
    )(y2, dst_h, zeros128)


def kernel(x, edge_index, edge_attr, Wl, bl, Wr, br, We, att, bias, W_ea,
           b_ea, gn_weight, gn_bias, gn_mean_scale):
    zeros128 = jnp.zeros((_NP, 128), _f32)
    attc = att.reshape(128, 1)
    bl2 = bl.reshape(1, 128)
    br2 = br.reshape(1, 128)
    bias2 = bias.reshape(1, 128)
    gw2 = gn_weight.reshape(1, 128)
    gb2 = gn_bias.reshape(1, 128)
    gms2 = gn_mean_scale.reshape(1, 128)
    eidx_a = edge_index[:, :_EH]
    eidx_b = edge_index[:, _EH:]
    dst_a = edge_index[1, :_EH]
    dst_b = edge_index[1, _EH:]
    ea_a = edge_attr[:_EH]
    ea_b = edge_attr[_EH:]

    x_l, x_r = pl.pallas_call(
        _tc_xlr,
        out_shape=[jax.ShapeDtypeStruct((_N, 128), _f32)] * 2,
    )(x, Wl, bl2, Wr, br2)

    gl_a, gr_a = _run_gather(x_l, x_r, eidx_a)
    gl_b, gr_b = _run_gather(x_l, x_r, eidx_b)
    y2_a = _run_mid(gl_a, gr_a, ea_a, We, attc)
    y2_b = _run_mid(gl_b, gr_b, ea_b, We, attc)
    parts_a = _run_scatter(y2_a, dst_a, zeros128)
    parts_b = _run_scatter(y2_b, dst_b, zeros128)
    out = pl.pallas_call(
        _tc_post,
        out_shape=jax.ShapeDtypeStruct((_N, 128), _f32),
    )(parts_a[:, :_N], parts_b[:, :_N], x_l, x_r, x, We, attc, bias2, gw2,
      gb2, gms2)
    return out


# speedup vs baseline: 54.6813x; 1.0024x over previous
import jax
import jax.numpy as jnp
from jax import lax
from jax.experimental import pallas as pl
from jax.experimental.pallas import tpu as pltpu
from jax.experimental.pallas import tpu_sc as plsc

_N = 10000
_E = 320000
_D = 128
_H = 4
_C = 32

_NC = 2
_NS = 16
_NW = _NC * _NS
_K = 128

_NCH = 78
_EPW = _NCH * _K
_XBASE = _EPW * _NW

_EPW2 = _E // _NS
_NCH2 = _EPW2 // _K
_TAIL2 = _EPW2 - _NCH2 * _K

_NP = 10240
_RPT = _NP // _NS

_f32 = jnp.float32


def _mm(a, b):
    return lax.dot_general(a, b, (((1,), (0,)), ((), ())),
                           preferred_element_type=_f32)


def _mmT(a, b):
    return lax.dot_general(a, b, (((1,), (1,)), ((), ())),
                           preferred_element_type=_f32)


def _sel16x128():
    rr = lax.broadcasted_iota(jnp.int32, (16, 128), 0)
    cc = lax.broadcasted_iota(jnp.int32, (16, 128), 1)
    return ((cc // _C) == rr).astype(_f32)


def _ag(attc):
    jr = lax.broadcasted_iota(jnp.int32, (128, 16), 0)
    hc = lax.broadcasted_iota(jnp.int32, (128, 16), 1)
    return ((jr // _C) == hc).astype(_f32) * attc


def _tc_xlr(x_ref, wl_ref, bl_ref, wr_ref, br_ref, xl_out, xr_out):
    xv = x_ref[...]
    xl_out[...] = _mmT(xv, wl_ref[...]) + bl_ref[...]
    xr_out[...] = _mmT(xv, wr_ref[...]) + br_ref[...]


def _tc_mid(gl_ref, gr_ref, ea_ref, we_ref, attc_ref, y2_out):
    gl = gl_ref[...]
    ea = ea_ref[...]
    ee = _mmT(ea, we_ref[...])
    m = gl + gr_ref[...] + ee
    m = jnp.maximum(m, 0.2 * m)
    l16 = _mm(m, _ag(attc_ref[...]))
    e16 = jnp.exp(l16)
    wex = _mm(e16, _sel16x128())
    r1 = lax.broadcasted_iota(jnp.int32, (16, 128), 0)
    c1 = lax.broadcasted_iota(jnp.int32, (16, 128), 1)
    Z1 = ((r1 == c1) & (r1 < 5)).astype(_f32)
    r2 = lax.broadcasted_iota(jnp.int32, (4, 128), 0)
    c2 = lax.broadcasted_iota(jnp.int32, (4, 128), 1)
    Z2 = (c2 == (r2 + 5)).astype(_f32)
    y2_out[0] = wex * gl
    y2_out[1] = _mm(e16, Z1) + _mm(ea, Z2)


def _tc_post(p_ref, xl_ref, xr_ref, x_ref, we_ref,
             attc_ref, bias_ref, gw_ref, gb_ref, gms_ref, o_ref):
    p = p_ref[...]
    acc = p[0]
    accz = p[1]
    ri = lax.broadcasted_iota(jnp.int32, (128, 128), 0)
    ci = lax.broadcasted_iota(jnp.int32, (128, 128), 1)
    denS = ((ci // _C) == ri).astype(_f32)
    r4 = lax.broadcasted_iota(jnp.int32, (128, 4), 0)
    c4 = lax.broadcasted_iota(jnp.int32, (128, 4), 1)
    D4 = (r4 == 4).astype(_f32)
    E4 = (r4 == (c4 + 5)).astype(_f32)
    den_part = _mm(accz, denS)
    deg4 = _mm(accz, D4)
    easum4 = _mm(accz, E4)
    la4 = easum4 / jnp.maximum(deg4, 1.0)
    ee = _mmT(la4, we_ref[...])
    xl = xl_ref[...]
    m = xl + xr_ref[...] + ee
    m = jnp.maximum(m, 0.2 * m)
    ls16 = _mm(m, _ag(attc_ref[...]))
    esl = jnp.exp(ls16)
    eslx = _mm(esl, _sel16x128())
    den = den_part + eslx
    out = (acc + eslx * xl) / den
    out = out + bias_ref[...] + x_ref[...]
    mean = jnp.mean(out, axis=0, keepdims=True)
    oc = out - mean * gms_ref[...]
    var = jnp.mean(oc * oc, axis=0, keepdims=True)
    outn = gw_ref[...] * oc / jnp.sqrt(var + 1e-5) + gb_ref[...]
    o_ref[...] = jnp.where(outn > 0.0, outn,
                           jnp.exp(jnp.minimum(outn, 0.0)) - 1.0)


_NB = 3
_NTRI = _NCH // _NB


def _make_gather(nch, nxtra, xbase):
    ntri = nch // _NB
    epw = nch * _K

    def gather(xl, xr, eidx, gl, gr,
               ix0, ix1, ix2, glv0, grv0, glv1, grv1, glv2, grv2,
               sg0, sg1, sg2, ss0, ss1, ss2):
        cid = lax.axis_index("c")
        sid = lax.axis_index("s")
        wid = sid * _NC + cid
        base = wid * epw
        bufs = ((ix0, glv0, grv0, sg0, ss0),
                (ix1, glv1, grv1, sg1, ss1),
                (ix2, glv2, grv2, sg2, ss2))

        def load_idx(j, ix):
            off = base + j * _K
            pltpu.sync_copy(eidx.at[:, pl.ds(off, _K)], ix)

        def start_gather(ix, bl, br_, sem):
            pltpu.async_copy(xl.at[ix.at[0]], bl, sem)
            pltpu.async_copy(xr.at[ix.at[1]], br_, sem)

        def wait_gather(bl, br_, sem):
            pltpu.make_async_copy(xl.at[pl.ds(0, _K)], bl, sem).wait()
            pltpu.make_async_copy(xr.at[pl.ds(0, _K)], br_, sem).wait()

        def start_store(j, bl, br_, sem):
            off = base + j * _K
            pltpu.async_copy(bl, gl.at[pl.ds(off, _K)], sem)
            pltpu.async_copy(br_, gr.at[pl.ds(off, _K)], sem)

        def wait_store(bl, br_, sem):
            pltpu.make_async_copy(bl, gl.at[pl.ds(0, _K)], sem).wait()
            pltpu.make_async_copy(br_, gr.at[pl.ds(0, _K)], sem).wait()

        for b, (ix, bl, br_, sg, ss) in enumerate(bufs):
            load_idx(b, ix)
            start_gather(ix, bl, br_, sg)
        for b, (ix, bl, br_, sg, ss) in enumerate(bufs):
            wait_gather(bl, br_, sg)
            start_store(b, bl, br_, ss)

        def rot(t, carry):
            j0 = _NB * t
            for b, (ix, bl, br_, sg, ss) in enumerate(bufs):
                wait_store(bl, br_, ss)
                load_idx(j0 + b, ix)
                start_gather(ix, bl, br_, sg)
            for b, (ix, bl, br_, sg, ss) in enumerate(bufs):
                wait_gather(bl, br_, sg)
                start_store(j0 + b, bl, br_, ss)
            return carry

        lax.fori_loop(1, ntri, rot, 0)
        for ix, bl, br_, sg, ss in bufs:
            wait_store(bl, br_, ss)

        @pl.when(wid < nxtra)
        def _extra():
            offx = xbase + wid * _K
            pltpu.sync_copy(eidx.at[:, pl.ds(offx, _K)], ix0)
            c0 = pltpu.async_copy(xl.at[ix0.at[0]], glv0, sg0)
            c1 = pltpu.async_copy(xr.at[ix0.at[1]], grv0, sg0)
            c0.wait()
            c1.wait()
            pltpu.sync_copy(glv0, gl.at[pl.ds(offx, _K)])
            pltpu.sync_copy(grv0, gr.at[pl.ds(offx, _K)])

    return gather


def _make_scatter(nch2, tail2):
    epw2 = nch2 * _K + tail2

    def scatter(y2, dst, zeros128, out2,
                idx0, idx1, idx_t, yv0, yv1, yt,
                sy0, sy1, sc0, sc1, acc_sh):
        cid = lax.axis_index("c")
        sid = lax.axis_index("s")
        base = sid * epw2
        row0 = sid * _RPT
        pltpu.sync_copy(zeros128.at[pl.ds(row0, _RPT)],
                        acc_sh.at[pl.ds(row0, _RPT)])
        plsc.subcore_barrier()
        bufs = ((idx0, yv0, sy0, sc0), (idx1, yv1, sy1, sc1))

        def load(j, ix, yv, sem):
            off = base + j * _K
            pltpu.sync_copy(dst.at[pl.ds(off, _K)], ix)
            pltpu.async_copy(y2.at[cid, pl.ds(off, _K)], yv, sem)

        def wait_load(yv, sem):
            pltpu.make_async_copy(y2.at[cid, pl.ds(0, _K)], yv, sem).wait()

        def start_scatter(ix, yv, sem):
            pltpu.async_copy(yv, acc_sh.at[ix], sem, add=True)

        def wait_scatter(yv, sem):
            pltpu.make_async_copy(yv, acc_sh.at[pl.ds(0, _K)], sem).wait()

        for b, (ix, yv, sy, sc) in enumerate(bufs):
            load(b, ix, yv, sy)
        for b, (ix, yv, sy, sc) in enumerate(bufs):
            wait_load(yv, sy)
            start_scatter(ix, yv, sc)

        def rot(t, carry):
            j0 = 2 * t
            for b, (ix, yv, sy, sc) in enumerate(bufs):
                wait_scatter(yv, sc)
                load(j0 + b, ix, yv, sy)
            for b, (ix, yv, sy, sc) in enumerate(bufs):
                wait_load(yv, sy)
                start_scatter(ix, yv, sc)
            return carry

        lax.fori_loop(1, nch2 // 2, rot, 0)
        for ix, yv, sy, sc in bufs:
            wait_scatter(yv, sc)

        offt = base + nch2 * _K
        pltpu.sync_copy(dst.at[pl.ds(offt, tail2)], idx_t)
        pltpu.sync_copy(y2.at[cid, pl.ds(offt, tail2)], yt)
        pltpu.sync_copy(yt, acc_sh.at[idx_t], add=True)
        plsc.subcore_barrier()
        pltpu.sync_copy(acc_sh.at[pl.ds(row0, _RPT)],
                        out2.at[cid, pl.ds(row0, _RPT)])

    return scatter


def _sc_mesh():
    return plsc.VectorSubcoreMesh(core_axis_name="c", subcore_axis_name="s")


_BE = 8000
_EH = _E // 2
_NBLKH = _EH // _BE
_NCHH = 39
_XTRA = 2
_XBASEH = _NCHH * _K * _NW
_NCH2H = 78
_TAIL2H = 16

_gather_half = _make_gather(_NCHH, _XTRA, _XBASEH)
_scatter_half = _make_scatter(_NCH2H, _TAIL2H)

_GATHER_SCRATCH = [
    pltpu.VMEM((2, _K), jnp.int32),
    pltpu.VMEM((2, _K), jnp.int32),
    pltpu.VMEM((2, _K), jnp.int32),
    pltpu.VMEM((_K, 128), _f32),
    pltpu.VMEM((_K, 128), _f32),
    pltpu.VMEM((_K, 128), _f32),
    pltpu.VMEM((_K, 128), _f32),
    pltpu.VMEM((_K, 128), _f32),
    pltpu.VMEM((_K, 128), _f32),
    pltpu.SemaphoreType.DMA,
    pltpu.SemaphoreType.DMA,
    pltpu.SemaphoreType.DMA,
    pltpu.SemaphoreType.DMA,
    pltpu.SemaphoreType.DMA,
    pltpu.SemaphoreType.DMA,
]

_SCATTER_SCRATCH = [
    pltpu.VMEM((_K,), jnp.int32),
    pltpu.VMEM((_K,), jnp.int32),
    pltpu.VMEM((_TAIL2H,), jnp.int32),
    pltpu.VMEM((_K, 128), _f32),
    pltpu.VMEM((_K, 128), _f32),
    pltpu.VMEM((_TAIL2H, 128), _f32),
    pltpu.SemaphoreType.DMA,
    pltpu.SemaphoreType.DMA,
    pltpu.SemaphoreType.DMA,
    pltpu.SemaphoreType.DMA,
    pltpu.VMEM_SHARED((_NP, 128), _f32),
]


def _run_gather(x_l, x_r, eidx_h):
    return pl.kernel(
        _gather_half,
        out_type=(
            jax.ShapeDtypeStruct((_EH, 128), _f32),
            jax.ShapeDtypeStruct((_EH, 128), _f32),
        ),
        mesh=_sc_mesh(),
        scratch_types=_GATHER_SCRATCH,
    )(x_l, x_r, eidx_h)


def _run_mid(gl, gr, ea_h, We, attc):
    return pl.pallas_call(
        _tc_mid,
        grid=(_NBLKH,),
        in_specs=[
            pl.BlockSpec((_BE, 128), lambda i: (i, 0)),
            pl.BlockSpec((_BE, 128), lambda i: (i, 0)),
            pl.BlockSpec((_BE, 4), lambda i: (i, 0)),
            pl.BlockSpec((128, 4), lambda i: (0, 0)),
            pl.BlockSpec((128, 1), lambda i: (0, 0)),
        ],
        out_specs=pl.BlockSpec((2, _BE, 128), lambda i: (0, i, 0)),
        out_shape=jax.ShapeDtypeStruct((2, _EH, 128), _f32),
    )(gl, gr, ea_h, We, attc)


def _run_scatter(y2, dst_h, zeros128):
    return pl.kernel(
        _scatter_half,
        out_type=jax.ShapeDtypeStruct((2, _NP, 128), _f32),
        mesh=_sc_mesh(),
        scratch_types=_SCATTER_SCRATCH,
    )(y2, dst_h, zeros128)


def kernel(x, edge_index, edge_attr, Wl, bl, Wr, br, We, att, bias, W_ea,
           b_ea, gn_weight, gn_bias, gn_mean_scale):
    zeros128 = jnp.zeros((_NP, 128), _f32)
    attc = att.reshape(128, 1)
    bl2 = bl.reshape(1, 128)
    br2 = br.reshape(1, 128)
    bias2 = bias.reshape(1, 128)
    gw2 = gn_weight.reshape(1, 128)
    gb2 = gn_bias.reshape(1, 128)
    gms2 = gn_mean_scale.reshape(1, 128)
    eidx_a = edge_index[:, :_EH]
    eidx_b = edge_index[:, _EH:]
    dst_a = edge_index[1, :_EH]
    dst_b = edge_index[1, _EH:]
    ea_a = edge_attr[:_EH]
    ea_b = edge_attr[_EH:]

    x_l, x_r = pl.pallas_call(
        _tc_xlr,
        out_shape=[jax.ShapeDtypeStruct((_N, 128), _f32)] * 2,
    )(x, Wl, bl2, Wr, br2)

    gl_a, gr_a = _run_gather(x_l, x_r, eidx_a)
    gl_b, gr_b = _run_gather(x_l, x_r, eidx_b)
    y2_a = _run_mid(gl_a, gr_a, ea_a, We, attc)
    y2_b = _run_mid(gl_b, gr_b, ea_b, We, attc)
    parts_a = _run_scatter(y2_a, dst_a, zeros128)
    parts_b = _run_scatter(y2_b, dst_b, zeros128)
    out = pl.pallas_call(
        _tc_post,
        out_shape=jax.ShapeDtypeStruct((_N, 128), _f32),
    )(parts_a[:, :_N] + parts_b[:, :_N],
      x_l, x_r, x, We, attc, bias2, gw2, gb2, gms2)
    return out
